# baseline (speedup 1.0000x reference)
import functools

import jax
import jax.numpy as jnp
from jax import lax
from jax.experimental import pallas as pl
from jax.experimental.pallas import tpu as pltpu
from jax.experimental.pallas import tpu_sc as plsc

N = 10000
E = 320000
FEAT = 128
HID = 32
EMB = 16

NC = 2
NS = 16
NW = NC * NS
CH = 128
KCH = 80
EPAD = NW * KCH * CH
NPAD = 10240
RPW = NPAD // NS
NB = 5

_mesh = functools.partial(
    plsc.VectorSubcoreMesh,
    core_axis_name="c",
    subcore_axis_name="s",
    num_cores=NC,
    num_subcores=NS,
)
_SC_PARAMS = pltpu.CompilerParams(use_tc_tiling_on_sc=False)


@functools.partial(
    pl.kernel,
    out_type=jax.ShapeDtypeStruct((NC * NPAD, 1), jnp.float32),
    mesh=_mesh(),
    compiler_params=_SC_PARAMS,
    scratch_types=[
        pltpu.VMEM((KCH, CH), jnp.int32),
        pltpu.VMEM((CH, 1), jnp.float32),
        pltpu.VMEM_SHARED((NPAD, 1), jnp.float32),
        pltpu.SemaphoreType.DMA,
    ],
)
def _deg_kernel(dst_hbm, ones_hbm, zero_hbm, out_hbm, dst_v, ones_v, deg_sh,
                sem):
    cid = lax.axis_index("c")
    sid = lax.axis_index("s")
    w = cid * NS + sid
    pltpu.sync_copy(dst_hbm.at[pl.ds(w * KCH, KCH)], dst_v)
    pltpu.sync_copy(ones_hbm, ones_v)
    pltpu.sync_copy(zero_hbm.at[pl.ds(sid * RPW, RPW)],
                    deg_sh.at[pl.ds(sid * RPW, RPW)])
    plsc.subcore_barrier()

    def body(c, carry):
        pltpu.sync_copy(ones_v, deg_sh.at[dst_v.at[c]], add=True)
        return carry

    lax.fori_loop(0, KCH, body, 0)
    plsc.subcore_barrier()
    pltpu.sync_copy(deg_sh.at[pl.ds(sid * RPW, RPW)],
                    out_hbm.at[pl.ds(cid * NPAD + sid * RPW, RPW)])


def _make_agg_kernel(D):
    @functools.partial(
        pl.kernel,
        out_type=jax.ShapeDtypeStruct((NC * NPAD, D), jnp.float32),
        mesh=_mesh(),
        compiler_params=_SC_PARAMS,
        scratch_types=[
            pltpu.VMEM((KCH, CH), jnp.int32),
            pltpu.VMEM((KCH, CH), jnp.int32),
        ] + [pltpu.VMEM((CH, D), jnp.float32)] * NB + [
            pltpu.VMEM_SHARED((NPAD, D), jnp.float32),
        ] + [pltpu.SemaphoreType.DMA] * NB,
    )
    def agg_kernel(hs_hbm, src_hbm, dst_hbm, zero_hbm, out_hbm,
                   src_v, dst_v, *rest):
        rows_b = rest[:NB]
        agg_sh = rest[NB]
        sems = rest[NB + 1:]
        cid = lax.axis_index("c")
        sid = lax.axis_index("s")
        w = cid * NS + sid
        pltpu.sync_copy(src_hbm.at[pl.ds(w * KCH, KCH)], src_v)
        pltpu.sync_copy(dst_hbm.at[pl.ds(w * KCH, KCH)], dst_v)
        pltpu.sync_copy(zero_hbm.at[pl.ds(sid * RPW, RPW)],
                        agg_sh.at[pl.ds(sid * RPW, RPW)])
        plsc.subcore_barrier()

        def group(j, carry):
            c0 = j * NB
            descs = [
                pltpu.async_copy(
                    hs_hbm.at[src_v.at[c0 + b]], rows_b[b], sems[b])
                for b in range(NB)
            ]
            for b in range(NB):
                descs[b].wait()
                pltpu.sync_copy(rows_b[b], agg_sh.at[dst_v.at[c0 + b]],
                                add=True)
            return carry

        lax.fori_loop(0, KCH // NB, group, 0)
        plsc.subcore_barrier()
        pltpu.sync_copy(agg_sh.at[pl.ds(sid * RPW, RPW)],
                        out_hbm.at[pl.ds(cid * NPAD + sid * RPW, RPW)])

    return agg_kernel


_agg32 = _make_agg_kernel(HID)
_agg16 = _make_agg_kernel(EMB)


def _k1_body(degp_ref, x_ref, w1_ref, h1s_ref, dinv_ref):
    degp = degp_ref[...]
    deg = degp[0] + degp[1] + 1.0
    dinv = lax.rsqrt(deg)
    h1 = jnp.dot(x_ref[...], w1_ref[...], preferred_element_type=jnp.float32)
    h1s_ref[...] = h1 * dinv
    dinv_ref[...] = dinv


def _k2_body(aggp_ref, h1s_ref, dinv_ref, b1_ref, w2_ref, h2s_ref):
    aggp = aggp_ref[...]
    dinv = dinv_ref[...]
    a = aggp[0] + aggp[1] + h1s_ref[...]
    out1 = jnp.maximum(a * dinv + b1_ref[...], 0.0)
    h2 = jnp.dot(out1, w2_ref[...], preferred_element_type=jnp.float32)
    h2s_ref[...] = h2 * dinv


def _k3_body(aggp_ref, h2s_ref, dinv_ref, b2_ref, out_ref):
    aggp = aggp_ref[...]
    out_ref[...] = (aggp[0] + aggp[1] + h2s_ref[...]) * dinv_ref[...] \
        + b2_ref[...]


_k1 = pl.pallas_call(
    _k1_body,
    out_shape=(jax.ShapeDtypeStruct((NPAD, HID), jnp.float32),
               jax.ShapeDtypeStruct((NPAD, 1), jnp.float32)),
)

_k2 = pl.pallas_call(
    _k2_body,
    out_shape=jax.ShapeDtypeStruct((NPAD, EMB), jnp.float32),
)

_k3 = pl.pallas_call(
    _k3_body,
    out_shape=jax.ShapeDtypeStruct((NPAD, EMB), jnp.float32),
)


def kernel(x, ei, mask_new, mask_old, embeds, W1, b1, W2, b2):
    pad = EPAD - E
    src_r = jnp.concatenate(
        [ei[0], jnp.zeros((pad,), jnp.int32)]).reshape(NW * KCH, CH)
    dst_r = jnp.concatenate(
        [ei[1], jnp.full((pad,), N, jnp.int32)]).reshape(NW * KCH, CH)
    xp = jnp.pad(x, ((0, NPAD - N), (0, 0)))
    ones_col = jnp.ones((CH, 1), jnp.float32)
    zeros1 = jnp.zeros((NPAD, 1), jnp.float32)
    zeros32 = jnp.zeros((NPAD, HID), jnp.float32)
    zeros16 = jnp.zeros((NPAD, EMB), jnp.float32)

    degp = _deg_kernel(dst_r, ones_col, zeros1)
    h1s, dinv = _k1(degp.reshape(NC, NPAD, 1), xp, W1)
    agg1 = _agg32(h1s, src_r, dst_r, zeros32)
    h2s = _k2(agg1.reshape(NC, NPAD, HID), h1s, dinv, b1.reshape(1, HID), W2)
    agg2 = _agg16(h2s, src_r, dst_r, zeros16)
    out = _k3(agg2.reshape(NC, NPAD, EMB), h2s, dinv, b2.reshape(1, EMB))
    return out[:N]


# speedup vs baseline: 1.0438x; 1.0438x over previous
import functools

import jax
import jax.numpy as jnp
from jax import lax
from jax.experimental import pallas as pl
from jax.experimental.pallas import tpu as pltpu
from jax.experimental.pallas import tpu_sc as plsc

N = 10000
E = 320000
FEAT = 128
HID = 32
EMB = 16

NC = 2
NS = 16
NW = NC * NS
CH = 128
KCH = 80
EPAD = NW * KCH * CH
NPAD = 10240
RPW = NPAD // NS
NB = 5

_mesh = functools.partial(
    plsc.VectorSubcoreMesh,
    core_axis_name="c",
    subcore_axis_name="s",
    num_cores=NC,
    num_subcores=NS,
)
_SC_PARAMS = pltpu.CompilerParams(use_tc_tiling_on_sc=False)


@functools.partial(
    pl.kernel,
    out_type=jax.ShapeDtypeStruct((NC * NPAD, 1), jnp.float32),
    mesh=_mesh(),
    compiler_params=_SC_PARAMS,
    scratch_types=[
        pltpu.VMEM((KCH, CH), jnp.int32),
        pltpu.VMEM((CH, 1), jnp.float32),
        pltpu.VMEM_SHARED((NPAD, 1), jnp.float32),
        pltpu.SemaphoreType.DMA,
    ],
)
def _deg_kernel(dst_hbm, ones_hbm, zero_hbm, out_hbm, dst_v, ones_v, deg_sh,
                sem):
    cid = lax.axis_index("c")
    sid = lax.axis_index("s")
    w = cid * NS + sid
    pltpu.sync_copy(dst_hbm.at[pl.ds(w * KCH, KCH)], dst_v)
    pltpu.sync_copy(ones_hbm, ones_v)
    pltpu.sync_copy(zero_hbm.at[pl.ds(sid * RPW, RPW)],
                    deg_sh.at[pl.ds(sid * RPW, RPW)])
    plsc.subcore_barrier()

    def body(c, carry):
        pltpu.sync_copy(ones_v, deg_sh.at[dst_v.at[c]], add=True)
        return carry

    lax.fori_loop(0, KCH, body, 0)
    plsc.subcore_barrier()
    pltpu.sync_copy(deg_sh.at[pl.ds(sid * RPW, RPW)],
                    out_hbm.at[pl.ds(cid * NPAD + sid * RPW, RPW)])


def _make_agg_kernel(D):
    @functools.partial(
        pl.kernel,
        out_type=jax.ShapeDtypeStruct((NC * NPAD, D), jnp.float32),
        mesh=_mesh(),
        compiler_params=_SC_PARAMS,
        scratch_types=[
            pltpu.VMEM((KCH, CH), jnp.int32),
            pltpu.VMEM((KCH, CH), jnp.int32),
        ] + [pltpu.VMEM((CH, D), jnp.float32)] * NB + [
            pltpu.VMEM_SHARED((NPAD, D), jnp.float32),
        ] + [pltpu.SemaphoreType.DMA] * NB,
    )
    def agg_kernel(hs_hbm, src_hbm, dst_hbm, zero_hbm, out_hbm,
                   src_v, dst_v, *rest):
        rows_b = rest[:NB]
        agg_sh = rest[NB]
        sems = rest[NB + 1:]
        cid = lax.axis_index("c")
        sid = lax.axis_index("s")
        w = cid * NS + sid
        pltpu.sync_copy(src_hbm.at[pl.ds(w * KCH, KCH)], src_v)
        pltpu.sync_copy(dst_hbm.at[pl.ds(w * KCH, KCH)], dst_v)
        pltpu.sync_copy(zero_hbm.at[pl.ds(sid * RPW, RPW)],
                        agg_sh.at[pl.ds(sid * RPW, RPW)])
        plsc.subcore_barrier()

        for b in range(NB):
            pltpu.async_copy(hs_hbm.at[src_v.at[b]], rows_b[b], sems[b])

        def group(j, carry):
            c0 = j * NB
            for b in range(NB):
                c = c0 + b
                pltpu.make_async_copy(
                    hs_hbm.at[src_v.at[c]], rows_b[b], sems[b]).wait()
                pltpu.sync_copy(rows_b[b], agg_sh.at[dst_v.at[c]],
                                add=True)
                pltpu.async_copy(
                    hs_hbm.at[src_v.at[c + NB]], rows_b[b], sems[b])
            return carry

        lax.fori_loop(0, KCH // NB - 1, group, 0)
        c0 = KCH - NB
        for b in range(NB):
            c = c0 + b
            pltpu.make_async_copy(
                hs_hbm.at[src_v.at[c]], rows_b[b], sems[b]).wait()
            pltpu.sync_copy(rows_b[b], agg_sh.at[dst_v.at[c]], add=True)
        plsc.subcore_barrier()
        pltpu.sync_copy(agg_sh.at[pl.ds(sid * RPW, RPW)],
                        out_hbm.at[pl.ds(cid * NPAD + sid * RPW, RPW)])

    return agg_kernel


_agg32 = _make_agg_kernel(HID)
_agg16 = _make_agg_kernel(EMB)


def _k1_body(degp_ref, x_ref, w1_ref, h1s_ref, dinv_ref):
    degp = degp_ref[...]
    deg = degp[0] + degp[1] + 1.0
    dinv = lax.rsqrt(deg)
    h1 = jnp.dot(x_ref[...], w1_ref[...], preferred_element_type=jnp.float32)
    h1s_ref[...] = h1 * dinv
    dinv_ref[...] = dinv


def _k2_body(aggp_ref, h1s_ref, dinv_ref, b1_ref, w2_ref, h2s_ref):
    aggp = aggp_ref[...]
    dinv = dinv_ref[...]
    a = aggp[0] + aggp[1] + h1s_ref[...]
    out1 = jnp.maximum(a * dinv + b1_ref[...], 0.0)
    h2 = jnp.dot(out1, w2_ref[...], preferred_element_type=jnp.float32)
    h2s_ref[...] = h2 * dinv


def _k3_body(aggp_ref, h2s_ref, dinv_ref, b2_ref, out_ref):
    aggp = aggp_ref[...]
    out_ref[...] = (aggp[0] + aggp[1] + h2s_ref[...]) * dinv_ref[...] \
        + b2_ref[...]


_k1 = pl.pallas_call(
    _k1_body,
    out_shape=(jax.ShapeDtypeStruct((NPAD, HID), jnp.float32),
               jax.ShapeDtypeStruct((NPAD, 1), jnp.float32)),
)

_k2 = pl.pallas_call(
    _k2_body,
    out_shape=jax.ShapeDtypeStruct((NPAD, EMB), jnp.float32),
)

_k3 = pl.pallas_call(
    _k3_body,
    out_shape=jax.ShapeDtypeStruct((NPAD, EMB), jnp.float32),
)


def kernel(x, ei, mask_new, mask_old, embeds, W1, b1, W2, b2):
    pad = EPAD - E
    src_r = jnp.concatenate(
        [ei[0], jnp.zeros((pad,), jnp.int32)]).reshape(NW * KCH, CH)
    dst_r = jnp.concatenate(
        [ei[1], jnp.full((pad,), N, jnp.int32)]).reshape(NW * KCH, CH)
    xp = jnp.pad(x, ((0, NPAD - N), (0, 0)))
    ones_col = jnp.ones((CH, 1), jnp.float32)
    zeros1 = jnp.zeros((NPAD, 1), jnp.float32)
    zeros32 = jnp.zeros((NPAD, HID), jnp.float32)
    zeros16 = jnp.zeros((NPAD, EMB), jnp.float32)

    degp = _deg_kernel(dst_r, ones_col, zeros1)
    h1s, dinv = _k1(degp.reshape(NC, NPAD, 1), xp, W1)
    agg1 = _agg32(h1s, src_r, dst_r, zeros32)
    h2s = _k2(agg1.reshape(NC, NPAD, HID), h1s, dinv, b1.reshape(1, HID), W2)
    agg2 = _agg16(h2s, src_r, dst_r, zeros16)
    out = _k3(agg2.reshape(NC, NPAD, EMB), h2s, dinv, b2.reshape(1, EMB))
    return out[:N]


# speedup vs baseline: 1.0446x; 1.0007x over previous
import functools

import jax
import jax.numpy as jnp
from jax import lax
from jax.experimental import pallas as pl
from jax.experimental.pallas import tpu as pltpu
from jax.experimental.pallas import tpu_sc as plsc

N = 10000
E = 320000
FEAT = 128
HID = 32
EMB = 16

NC = 2
NS = 16
NW = NC * NS
CH = 128
KCH = 80
EPAD = NW * KCH * CH
NPAD = 10240
RPW = NPAD // NS
NB = 5

_mesh = functools.partial(
    plsc.VectorSubcoreMesh,
    core_axis_name="c",
    subcore_axis_name="s",
    num_cores=NC,
    num_subcores=NS,
)
_SC_PARAMS = pltpu.CompilerParams(use_tc_tiling_on_sc=False)


@functools.partial(
    pl.kernel,
    out_type=jax.ShapeDtypeStruct((NC * NPAD, 1), jnp.float32),
    mesh=_mesh(),
    compiler_params=_SC_PARAMS,
    scratch_types=[
        pltpu.VMEM((KCH, CH), jnp.int32),
        pltpu.VMEM((CH, 1), jnp.float32),
        pltpu.VMEM_SHARED((NPAD, 1), jnp.float32),
    ],
)
def _deg_kernel(dst_hbm, ones_hbm, zero_hbm, out_hbm, dst_v, ones_v, deg_sh):
    cid = lax.axis_index("c")
    sid = lax.axis_index("s")
    w = cid * NS + sid
    pltpu.sync_copy(dst_hbm.at[pl.ds(w * KCH, KCH)], dst_v)
    pltpu.sync_copy(ones_hbm, ones_v)
    pltpu.sync_copy(zero_hbm.at[pl.ds(sid * RPW, RPW)],
                    deg_sh.at[pl.ds(sid * RPW, RPW)])
    plsc.subcore_barrier()

    def body(c, carry):
        pltpu.sync_copy(ones_v, deg_sh.at[dst_v.at[c]], add=True)
        return carry

    lax.fori_loop(0, KCH, body, 0)
    plsc.subcore_barrier()
    pltpu.sync_copy(deg_sh.at[pl.ds(sid * RPW, RPW)],
                    out_hbm.at[pl.ds(cid * NPAD + sid * RPW, RPW)])


def _make_agg_kernel(D):
    @functools.partial(
        pl.kernel,
        out_type=jax.ShapeDtypeStruct((NC * NPAD, D), jnp.float32),
        mesh=_mesh(),
        compiler_params=_SC_PARAMS,
        scratch_types=[
            pltpu.VMEM((KCH, CH), jnp.int32),
            pltpu.VMEM((KCH, CH), jnp.int32),
        ] + [pltpu.VMEM((CH, D), jnp.float32)] * NB + [
            pltpu.VMEM_SHARED((NPAD, D), jnp.float32),
        ] + [pltpu.SemaphoreType.DMA] * NB,
    )
    def agg_kernel(hs_hbm, src_hbm, dst_hbm, zero_hbm, out_hbm,
                   src_v, dst_v, *rest):
        rows_b = rest[:NB]
        agg_sh = rest[NB]
        sems = rest[NB + 1:]
        cid = lax.axis_index("c")
        sid = lax.axis_index("s")
        w = cid * NS + sid
        pltpu.sync_copy(src_hbm.at[pl.ds(w * KCH, KCH)], src_v)
        pltpu.sync_copy(dst_hbm.at[pl.ds(w * KCH, KCH)], dst_v)
        pltpu.sync_copy(zero_hbm.at[pl.ds(sid * RPW, RPW)],
                        agg_sh.at[pl.ds(sid * RPW, RPW)])
        plsc.subcore_barrier()

        for b in range(NB):
            pltpu.async_copy(hs_hbm.at[src_v.at[b]], rows_b[b], sems[b])

        def group(j, carry):
            c0 = j * NB
            for b in range(NB):
                c = c0 + b
                pltpu.make_async_copy(
                    hs_hbm.at[src_v.at[c]], rows_b[b], sems[b]).wait()
                pltpu.sync_copy(rows_b[b], agg_sh.at[dst_v.at[c]],
                                add=True)
                pltpu.async_copy(
                    hs_hbm.at[src_v.at[c + NB]], rows_b[b], sems[b])
            return carry

        lax.fori_loop(0, KCH // NB - 1, group, 0)
        c0 = KCH - NB
        for b in range(NB):
            c = c0 + b
            pltpu.make_async_copy(
                hs_hbm.at[src_v.at[c]], rows_b[b], sems[b]).wait()
            pltpu.sync_copy(rows_b[b], agg_sh.at[dst_v.at[c]], add=True)

        plsc.subcore_barrier()
        pltpu.sync_copy(agg_sh.at[pl.ds(sid * RPW, RPW)],
                        out_hbm.at[pl.ds(cid * NPAD + sid * RPW, RPW)])

    return agg_kernel


_agg32 = _make_agg_kernel(HID)
_agg16 = _make_agg_kernel(EMB)


def _k1_body(degp_ref, x_ref, w1_ref, h1s_ref, dinv_ref):
    degp = degp_ref[...]
    deg = degp[0] + degp[1] + 1.0
    dinv = lax.rsqrt(deg)
    h1 = jnp.dot(x_ref[...], w1_ref[...], preferred_element_type=jnp.float32)
    h1s_ref[...] = h1 * dinv
    dinv_ref[...] = dinv


def _k2_body(aggp_ref, h1s_ref, dinv_ref, b1_ref, w2_ref, h2s_ref):
    aggp = aggp_ref[...]
    dinv = dinv_ref[...]
    a = aggp[0] + aggp[1] + h1s_ref[...]
    out1 = jnp.maximum(a * dinv + b1_ref[...], 0.0)
    h2 = jnp.dot(out1, w2_ref[...], preferred_element_type=jnp.float32)
    h2s_ref[...] = h2 * dinv


def _k3_body(aggp_ref, h2s_ref, dinv_ref, b2_ref, out_ref):
    aggp = aggp_ref[...]
    out_ref[...] = (aggp[0] + aggp[1] + h2s_ref[...]) * dinv_ref[...] \
        + b2_ref[...]


_k1 = pl.pallas_call(
    _k1_body,
    out_shape=(jax.ShapeDtypeStruct((NPAD, HID), jnp.float32),
               jax.ShapeDtypeStruct((NPAD, 1), jnp.float32)),
)

_k2 = pl.pallas_call(
    _k2_body,
    out_shape=jax.ShapeDtypeStruct((NPAD, EMB), jnp.float32),
)

_k3 = pl.pallas_call(
    _k3_body,
    out_shape=jax.ShapeDtypeStruct((NPAD, EMB), jnp.float32),
)


def kernel(x, ei, mask_new, mask_old, embeds, W1, b1, W2, b2):
    pad = EPAD - E
    src_r = jnp.concatenate(
        [ei[0], jnp.zeros((pad,), jnp.int32)]).reshape(NW * KCH, CH)
    dst_r = jnp.concatenate(
        [ei[1], jnp.full((pad,), N, jnp.int32)]).reshape(NW * KCH, CH)
    xp = jnp.pad(x, ((0, NPAD - N), (0, 0)))
    ones_col = jnp.ones((CH, 1), jnp.float32)
    zeros1 = jnp.zeros((NPAD, 1), jnp.float32)
    zeros32 = jnp.zeros((NPAD, HID), jnp.float32)
    zeros16 = jnp.zeros((NPAD, EMB), jnp.float32)

    degp = _deg_kernel(dst_r, ones_col, zeros1)
    h1s, dinv = _k1(degp.reshape(NC, NPAD, 1), xp, W1)
    agg1 = _agg32(h1s, src_r, dst_r, zeros32)
    h2s = _k2(agg1.reshape(NC, NPAD, HID), h1s, dinv, b1.reshape(1, HID), W2)
    agg2 = _agg16(h2s, src_r, dst_r, zeros16)
    out = _k3(agg2.reshape(NC, NPAD, EMB), h2s, dinv, b2.reshape(1, EMB))
    return out[:N]
